# Initial kernel scaffold; baseline (speedup 1.0000x reference)
import jax
import jax.numpy as jnp
from jax.experimental import pallas as pl


def kernel(x, conv1_w, conv1_b, nn1_w, nn1_b, conv2_w, conv2_b, nn2_w, nn2_b, fc1_w, fc1_b, fc2_w, fc2_b, samp1, samp2):
    raise NotImplementedError("write your pallas kernel here")


# speedup vs baseline: 50.2729x; 50.2729x over previous
import jax
import jax.numpy as jnp
from jax.experimental import pallas as pl
from jax.experimental.pallas import tpu as pltpu

_HW = 1024
_W = 32
_N = 64
_K = 9


def _layer(xf, C, O, wc, bc, wkn, bn, samp_row):
    f32 = jnp.float32
    xcol = jax.lax.broadcasted_iota(jnp.int32, (C, _HW), 1) % _W
    parts = []
    for dy in (-1, 0, 1):
        for dx in (-1, 0, 1):
            off = dy * _W + dx
            if off > 0:
                sh = jnp.concatenate(
                    [xf[:, off:], jnp.zeros((C, off), f32)], axis=1)
            elif off < 0:
                sh = jnp.concatenate(
                    [jnp.zeros((C, -off), f32), xf[:, :_HW + off]], axis=1)
            else:
                sh = xf
            if dx == -1:
                sh = jnp.where(xcol >= 1, sh, 0.0)
            elif dx == 1:
                sh = jnp.where(xcol <= _W - 2, sh, 0.0)
            parts.append(sh)
    x9 = jnp.concatenate(parts, axis=0)
    conv = jnp.dot(wc, x9, preferred_element_type=f32) + bc

    iota_p = jax.lax.broadcasted_iota(jnp.int32, (_HW, _N), 0)
    oh_samp = (iota_p == samp_row).astype(f32)
    s_t = jnp.dot(xf, oh_samp, preferred_element_type=f32, precision=jax.lax.Precision.HIGHEST)
    st_t = s_t.T
    xsq = jnp.sum(xf * xf, axis=0, keepdims=True)
    ssq = jnp.sum(st_t * st_t, axis=1, keepdims=True)
    m = jnp.dot(st_t, xf, preferred_element_type=f32)
    score = -(xsq - 2.0 * m + ssq)

    p2 = jnp.dot(wkn, s_t, preferred_element_type=f32)
    iota_n = jax.lax.broadcasted_iota(jnp.int32, (_N, _HW), 0)
    nn = jnp.zeros((O, _HW), f32)
    for k in range(_K):
        best = jnp.max(score, axis=0, keepdims=True)
        idx = jnp.min(jnp.where(score >= best, iota_n, _N),
                      axis=0, keepdims=True)
        sel = iota_n == idx
        oh = sel.astype(f32)
        score = jnp.where(sel, -1e30, score)
        nn = nn + jnp.dot(p2[k * O:(k + 1) * O], oh,
                          preferred_element_type=f32, precision=jax.lax.Precision.HIGHEST)
    nn = nn + bn
    return jnp.concatenate([conv, nn], axis=0)


def _feat_kernel(x_ref, w1c_ref, b1c_ref, wkn1_ref, b1n_ref,
                 w2c_ref, b2c_ref, wkn2_ref, b2n_ref,
                 s1_ref, s2_ref, out_ref):
    xf = x_ref[0]
    h1 = _layer(xf, 3, 8, w1c_ref[...], b1c_ref[...], wkn1_ref[...],
                b1n_ref[...], s1_ref[...])
    h1 = jnp.maximum(h1, 0.0)
    h2 = _layer(h1, 16, 16, w2c_ref[...], b2c_ref[...], wkn2_ref[...],
                b2n_ref[...], s2_ref[...])
    out_ref[0] = jnp.maximum(h2, 0.0)


_KB = 2048
_NKB = 32768 // _KB


def _fc_kernel(xb_ref, w1_ref, b1_ref, w2t_ref, b2_ref, out_ref, acc_ref):
    j = pl.program_id(0)

    @pl.when(j == 0)
    def _():
        acc_ref[...] = jnp.zeros_like(acc_ref)

    acc_ref[...] += jax.lax.dot_general(
        xb_ref[...], w1_ref[...], (((1,), (1,)), ((), ())),
        preferred_element_type=jnp.float32)

    @pl.when(j == _NKB - 1)
    def _():
        f = jnp.maximum(acc_ref[...] + b1_ref[...], 0.0)
        out_ref[...] = jnp.dot(f, w2t_ref[...],
                               preferred_element_type=jnp.float32) + b2_ref[...]


def kernel(x, conv1_w, conv1_b, nn1_w, nn1_b, conv2_w, conv2_b, nn2_w,
           nn2_b, fc1_w, fc1_b, fc2_w, fc2_b, samp1, samp2):
    B = x.shape[0]
    f32 = jnp.float32
    xr = x.reshape(B, 3, _HW)
    w1c = conv1_w.transpose(0, 2, 3, 1).reshape(8, 27)
    wkn1 = nn1_w.transpose(2, 0, 1).reshape(72, 3)
    w2c = conv2_w.transpose(0, 2, 3, 1).reshape(16, 144)
    wkn2 = nn2_w.transpose(2, 0, 1).reshape(144, 16)
    b1c = conv1_b.reshape(8, 1)
    b1n = nn1_b.reshape(8, 1)
    b2c = conv2_b.reshape(16, 1)
    b2n = nn2_b.reshape(16, 1)
    s1 = samp1.reshape(1, _N)
    s2 = samp2.reshape(1, _N)

    full = lambda shape: pl.BlockSpec(shape, lambda i: (0,) * len(shape))
    h2 = pl.pallas_call(
        _feat_kernel,
        grid=(B,),
        in_specs=[
            pl.BlockSpec((1, 3, _HW), lambda i: (i, 0, 0)),
            full((8, 27)), full((8, 1)), full((72, 3)), full((8, 1)),
            full((16, 144)), full((16, 1)), full((144, 16)), full((16, 1)),
            full((1, _N)), full((1, _N)),
        ],
        out_specs=pl.BlockSpec((1, 32, _HW), lambda i: (i, 0, 0)),
        out_shape=jax.ShapeDtypeStruct((B, 32, _HW), f32),
        compiler_params=pltpu.CompilerParams(
            dimension_semantics=("arbitrary",)),
    )(xr, w1c, b1c, wkn1, b1n, w2c, b2c, wkn2, b2n, s1, s2)

    feats = h2.reshape(B, 32 * _HW)
    out = pl.pallas_call(
        _fc_kernel,
        grid=(_NKB,),
        in_specs=[
            pl.BlockSpec((B, _KB), lambda j: (0, j)),
            pl.BlockSpec((1024, _KB), lambda j: (0, j)),
            full((1, 1024)), full((1024, 16)), full((1, 16)),
        ],
        out_specs=pl.BlockSpec((B, 16), lambda j: (0, 0)),
        out_shape=jax.ShapeDtypeStruct((B, 16), f32),
        scratch_shapes=[pltpu.VMEM((B, 1024), f32)],
        compiler_params=pltpu.CompilerParams(
            dimension_semantics=("arbitrary",)),
    )(feats, fc1_w, fc1_b.reshape(1, 1024),
      jnp.pad(fc2_w.T, ((0, 0), (0, 6))), jnp.pad(fc2_b, (0, 6)).reshape(1, 16))
    return out[:, :10]


# speedup vs baseline: 77.9479x; 1.5505x over previous
import jax
import jax.numpy as jnp
from jax.experimental import pallas as pl
from jax.experimental.pallas import tpu as pltpu

_HW = 1024
_W = 32
_N = 64
_K = 9


def _split3(x):
    bf = jnp.bfloat16
    f32 = jnp.float32
    a = x.astype(bf)
    r = x - a.astype(f32)
    b = r.astype(bf)
    c = (r - b.astype(f32)).astype(bf)
    return a, b, c


def _layer(xf, C, O, wc, bc, wkn, bn, samp_row):
    f32 = jnp.float32
    xcol = jax.lax.broadcasted_iota(jnp.int32, (C, _HW), 1) % _W
    parts = []
    for dy in (-1, 0, 1):
        for dx in (-1, 0, 1):
            off = dy * _W + dx
            if off > 0:
                sh = jnp.concatenate(
                    [xf[:, off:], jnp.zeros((C, off), f32)], axis=1)
            elif off < 0:
                sh = jnp.concatenate(
                    [jnp.zeros((C, -off), f32), xf[:, :_HW + off]], axis=1)
            else:
                sh = xf
            if dx == -1:
                sh = jnp.where(xcol >= 1, sh, 0.0)
            elif dx == 1:
                sh = jnp.where(xcol <= _W - 2, sh, 0.0)
            parts.append(sh)
    x9 = jnp.concatenate(parts, axis=0)
    conv = jnp.dot(wc, x9, preferred_element_type=f32) + bc

    bf = jnp.bfloat16
    iota_p = jax.lax.broadcasted_iota(jnp.int32, (_HW, _N), 0)
    oh_samp = (iota_p == samp_row).astype(f32).astype(bf)
    xa, xb, xc = _split3(xf)
    s_t = (jnp.dot(xa, oh_samp, preferred_element_type=f32)
           + jnp.dot(xb, oh_samp, preferred_element_type=f32)
           + jnp.dot(xc, oh_samp, preferred_element_type=f32))
    st_t = s_t.T
    xsq = jnp.sum(xf * xf, axis=0, keepdims=True)
    ssq = jnp.sum(st_t * st_t, axis=1, keepdims=True)
    m = jnp.dot(st_t, xf, preferred_element_type=f32)
    score = -(xsq - 2.0 * m + ssq)

    p2 = jnp.dot(wkn, s_t, preferred_element_type=f32)
    pa, pb, pc = _split3(p2)
    p2k = [jnp.concatenate([pa[k * O:(k + 1) * O],
                            pb[k * O:(k + 1) * O],
                            pc[k * O:(k + 1) * O]], axis=0)
           for k in range(_K)]
    iota_n = jax.lax.broadcasted_iota(jnp.int32, (_N, _HW), 0)
    nn = jnp.zeros((O, _HW), f32)
    for k in range(_K):
        best = jnp.max(score, axis=0, keepdims=True)
        idx = jnp.min(jnp.where(score >= best, iota_n, _N),
                      axis=0, keepdims=True)
        sel = iota_n == idx
        oh = sel.astype(f32).astype(bf)
        score = jnp.where(sel, -1e30, score)
        y = jnp.dot(p2k[k], oh, preferred_element_type=f32)
        nn = nn + y[:O] + y[O:2 * O] + y[2 * O:]
    nn = nn + bn
    return jnp.concatenate([conv, nn], axis=0)


_BB = 2


def _feat_kernel(x_ref, w1c_ref, b1c_ref, wkn1_ref, b1n_ref,
                 w2c_ref, b2c_ref, wkn2_ref, b2n_ref,
                 s1_ref, s2_ref, out_ref):
    for b in range(_BB):
        xf = x_ref[b]
        h1 = _layer(xf, 3, 8, w1c_ref[...], b1c_ref[...], wkn1_ref[...],
                    b1n_ref[...], s1_ref[...])
        h1 = jnp.maximum(h1, 0.0)
        h2 = _layer(h1, 16, 16, w2c_ref[...], b2c_ref[...], wkn2_ref[...],
                    b2n_ref[...], s2_ref[...])
        out_ref[b] = jnp.maximum(h2, 0.0)


_KB = 2048
_NKB = 32768 // _KB


def _fc_kernel(xb_ref, w1_ref, b1_ref, w2t_ref, b2_ref, out_ref, acc_ref):
    j = pl.program_id(0)

    @pl.when(j == 0)
    def _():
        acc_ref[...] = jnp.zeros_like(acc_ref)

    acc_ref[...] += jax.lax.dot_general(
        xb_ref[...], w1_ref[...], (((1,), (1,)), ((), ())),
        preferred_element_type=jnp.float32)

    @pl.when(j == _NKB - 1)
    def _():
        f = jnp.maximum(acc_ref[...] + b1_ref[...], 0.0)
        out_ref[...] = jnp.dot(f, w2t_ref[...],
                               preferred_element_type=jnp.float32) + b2_ref[...]


def kernel(x, conv1_w, conv1_b, nn1_w, nn1_b, conv2_w, conv2_b, nn2_w,
           nn2_b, fc1_w, fc1_b, fc2_w, fc2_b, samp1, samp2):
    B = x.shape[0]
    f32 = jnp.float32
    xr = x.reshape(B, 3, _HW)
    w1c = conv1_w.transpose(0, 2, 3, 1).reshape(8, 27)
    wkn1 = nn1_w.transpose(2, 0, 1).reshape(72, 3)
    w2c = conv2_w.transpose(0, 2, 3, 1).reshape(16, 144)
    wkn2 = nn2_w.transpose(2, 0, 1).reshape(144, 16)
    b1c = conv1_b.reshape(8, 1)
    b1n = nn1_b.reshape(8, 1)
    b2c = conv2_b.reshape(16, 1)
    b2n = nn2_b.reshape(16, 1)
    s1 = samp1.reshape(1, _N)
    s2 = samp2.reshape(1, _N)

    full = lambda shape: pl.BlockSpec(shape, lambda i: (0,) * len(shape))
    h2 = pl.pallas_call(
        _feat_kernel,
        grid=(B // _BB,),
        in_specs=[
            pl.BlockSpec((_BB, 3, _HW), lambda i: (i, 0, 0)),
            full((8, 27)), full((8, 1)), full((72, 3)), full((8, 1)),
            full((16, 144)), full((16, 1)), full((144, 16)), full((16, 1)),
            full((1, _N)), full((1, _N)),
        ],
        out_specs=pl.BlockSpec((_BB, 32, _HW), lambda i: (i, 0, 0)),
        out_shape=jax.ShapeDtypeStruct((B, 32, _HW), f32),
        compiler_params=pltpu.CompilerParams(
            dimension_semantics=("arbitrary",)),
    )(xr, w1c, b1c, wkn1, b1n, w2c, b2c, wkn2, b2n, s1, s2)

    feats = h2.reshape(B, 32 * _HW)
    out = pl.pallas_call(
        _fc_kernel,
        grid=(_NKB,),
        in_specs=[
            pl.BlockSpec((B, _KB), lambda j: (0, j)),
            pl.BlockSpec((1024, _KB), lambda j: (0, j)),
            full((1, 1024)), full((1024, 16)), full((1, 16)),
        ],
        out_specs=pl.BlockSpec((B, 16), lambda j: (0, 0)),
        out_shape=jax.ShapeDtypeStruct((B, 16), f32),
        scratch_shapes=[pltpu.VMEM((B, 1024), f32)],
        compiler_params=pltpu.CompilerParams(
            dimension_semantics=("arbitrary",)),
    )(feats, fc1_w, fc1_b.reshape(1, 1024),
      jnp.pad(fc2_w.T, ((0, 0), (0, 6))), jnp.pad(fc2_b, (0, 6)).reshape(1, 16))
    return out[:, :10]
